# baseline (speedup 1.0000x reference)
import functools

import numpy as np
import jax
import jax.numpy as jnp
from jax import lax
from jax.experimental import pallas as pl
from jax.experimental.pallas import tpu as pltpu
from jax.experimental.pallas import tpu_sc as plsc

N = 4096
T = 40
W = 4096
NSTEP = T - 1

_TINY = np.float32(np.finfo(np.float32).tiny)
_RANGE = np.float32(np.float32(1.0) - _TINY)


def _threefry_bits(k0, k1, idx):
    u32 = np.uint32
    ks2 = k0 ^ k1 ^ u32(0x1BD11BDA)

    def rounds(x0, x1, rots):
        for r in rots:
            x0 = x0 + x1
            x1 = (x1 << u32(r)) | (x1 >> u32(32 - r))
            x1 = x0 ^ x1
        return x0, x1

    r1 = (13, 15, 26, 6)
    r2 = (17, 29, 16, 24)
    x0 = jnp.zeros_like(idx) + k0
    x1 = idx + k1
    x0, x1 = rounds(x0, x1, r1)
    x0 = x0 + k1
    x1 = x1 + (ks2 + u32(1))
    x0, x1 = rounds(x0, x1, r2)
    x0 = x0 + ks2
    x1 = x1 + (k0 + u32(2))
    x0, x1 = rounds(x0, x1, r1)
    x0 = x0 + k0
    x1 = x1 + (k1 + u32(3))
    x0, x1 = rounds(x0, x1, r2)
    x0 = x0 + k1
    x1 = x1 + (ks2 + u32(4))
    x0, x1 = rounds(x0, x1, r1)
    x0 = x0 + ks2
    x1 = x1 + (k0 + u32(5))
    return x0 ^ x1


def _gumbel_from_bits(bits):
    fb = (bits >> np.uint32(9)) | np.uint32(0x3F800000)
    f = lax.bitcast_convert_type(fb, jnp.float32) - np.float32(1.0)
    u = jnp.maximum(_TINY, f * _RANGE + _TINY)
    return -jnp.log(-jnp.log(u))


_LOGP_BR = 256


def _logp_body(a_ref, dinv_ref, out_ref):
    out_ref[...] = jnp.log(a_ref[...] * dinv_ref[...] + np.float32(1e-30))


def _compute_logp(A, dinv):
    return pl.pallas_call(
        _logp_body,
        grid=(N // _LOGP_BR,),
        in_specs=[
            pl.BlockSpec((_LOGP_BR, N), lambda b: (b, 0)),
            pl.BlockSpec((_LOGP_BR, 1), lambda b: (b, 0)),
        ],
        out_specs=pl.BlockSpec((_LOGP_BR, N), lambda b: (b, 0)),
        out_shape=jax.ShapeDtypeStruct((N, N), jnp.float32),
    )(A, dinv)


_BW = 512
_CW = 512
_CB = 1024


def _sample_body(key_ref, rows_ref, bits_ref, next_ref, contrib_ref):
    b = pl.program_id(0)
    k0 = key_ref[0, 0]
    k1 = key_ref[0, 1]
    w0 = key_ref[0, 2] + (b * _BW).astype(jnp.uint32)

    run_max = jnp.full((_BW, 1), -jnp.inf, jnp.float32)
    run_arg = jnp.zeros((_BW, 1), jnp.int32)
    run_val = jnp.zeros((_BW, 1), jnp.float32)

    for c in range(N // _CW):
        row = rows_ref[:, c * _CW:(c + 1) * _CW]
        if (c + 1) * _CW <= _CB:
            bits = bits_ref[:, c * _CW:(c + 1) * _CW]
        else:
            ridx = lax.broadcasted_iota(jnp.uint32, (_BW, _CW), 0)
            cidx = lax.broadcasted_iota(jnp.uint32, (_BW, _CW), 1)
            idx = (w0 + ridx) * np.uint32(N) + (np.uint32(c * _CW) + cidx)
            bits = _threefry_bits(k0, k1, idx)
        g = _gumbel_from_bits(bits)
        vals = g + row
        cmax = jnp.max(vals, axis=1, keepdims=True)
        colix = lax.broadcasted_iota(jnp.int32, (_BW, _CW), 1) + np.int32(c * _CW)
        carg = jnp.min(
            jnp.where(vals == cmax, colix, np.int32(2147483647)),
            axis=1, keepdims=True)
        cval = jnp.sum(
            jnp.where(colix == carg, row, np.float32(0.0)),
            axis=1, keepdims=True)
        take = cmax > run_max
        run_arg = jnp.where(take, carg, run_arg)
        run_val = jnp.where(take, cval, run_val)
        run_max = jnp.maximum(run_max, cmax)

    next_ref[...] = run_arg
    contrib_ref[...] = run_val


def _sample_step(rows, bits, key3, wblk=0):
    B = rows.shape[0]
    return pl.pallas_call(
        _sample_body,
        grid=(B // _BW,),
        in_specs=[
            pl.BlockSpec(memory_space=pltpu.SMEM),
            pl.BlockSpec((_BW, N), lambda b: (b, 0)),
            pl.BlockSpec((_BW, _CB), lambda b, _w=wblk: (b + _w, 0)),
        ],
        out_specs=[
            pl.BlockSpec((_BW, 1), lambda b: (b, 0)),
            pl.BlockSpec((_BW, 1), lambda b: (b, 0)),
        ],
        out_shape=[
            jax.ShapeDtypeStruct((B, 1), jnp.int32),
            jax.ShapeDtypeStruct((B, 1), jnp.float32),
        ],
    )(key3, rows, bits)


_SC_NW = 32
_SC_CH = 8


def _sc_gather(table, idx):
    B = idx.shape[0]
    bpw = B // _SC_NW
    nch = bpw // _SC_CH
    mesh = plsc.VectorSubcoreMesh(core_axis_name="c", subcore_axis_name="s")

    @functools.partial(
        pl.kernel,
        mesh=mesh,
        out_type=jax.ShapeDtypeStruct((B, N), jnp.float32),
        scratch_types=[
            pltpu.VMEM((bpw,), jnp.int32),
            pltpu.VMEM((_SC_CH, N), jnp.float32),
            pltpu.VMEM((_SC_CH, N), jnp.float32),
            pltpu.SemaphoreType.DMA,
            pltpu.SemaphoreType.DMA,
        ],
    )
    def k(table_hbm, idx_hbm, out_hbm, idx_v, rows0, rows1, sem0, sem1):
        wid = lax.axis_index("s") * 2 + lax.axis_index("c")
        base = wid * bpw
        pltpu.sync_copy(idx_hbm.at[pl.ds(base, bpw)], idx_v)
        bufs = (rows0, rows1)
        sems = (sem0, sem1)

        def dma(i):
            return pltpu.make_async_copy(
                table_hbm.at[idx_v.at[pl.ds(i * _SC_CH, _SC_CH)]],
                bufs[i % 2], sems[i % 2])

        dma(0).start()
        for i in range(nch):
            if i + 1 < nch:
                dma(i + 1).start()
            dma(i).wait()
            pltpu.sync_copy(
                bufs[i % 2],
                out_hbm.at[pl.ds(base + i * _SC_CH, _SC_CH)])

    return k(table, idx)


_BITS_GW = 16


def _sc_bits(keyrep, dep):
    mesh = plsc.VectorSubcoreMesh(core_axis_name="c", subcore_axis_name="s")
    bpw = W // _SC_NW

    @functools.partial(
        pl.kernel,
        mesh=mesh,
        out_type=jax.ShapeDtypeStruct((W, _CB), jnp.uint32),
        scratch_types=[
            pltpu.VMEM((2, 16), jnp.uint32),
            pltpu.VMEM((_BITS_GW, _CB), jnp.uint32),
            pltpu.VMEM((16,), jnp.int32),
        ],
    )
    def k(keyrep_hbm, dep_hbm, out_hbm, kv, buf, depv):
        wid = lax.axis_index("s") * 2 + lax.axis_index("c")
        pltpu.sync_copy(dep_hbm.at[pl.ds(0, 16)], depv)
        pltpu.sync_copy(keyrep_hbm, kv)
        k0 = kv[0, :]
        k1 = kv[1, :]
        iota = lax.iota(jnp.int32, 16).astype(jnp.uint32)
        base_w = wid * bpw

        def grp(gi, carry0):
            def wloop(wl, carry1):
                w = base_w + gi * _BITS_GW + wl
                rowbase = (w * N).astype(jnp.uint32)

                def cloop(cv, carry2):
                    j0 = cv * 16
                    idx = jnp.full((16,), rowbase + j0.astype(jnp.uint32),
                                   jnp.uint32) + iota
                    buf[wl, pl.ds(j0, 16)] = _threefry_bits(k0, k1, idx)
                    return carry2

                lax.fori_loop(0, _CB // 16, cloop, 0)
                return carry1

            lax.fori_loop(0, _BITS_GW, wloop, 0)
            pltpu.sync_copy(
                buf, out_hbm.at[pl.ds(base_w + gi * _BITS_GW, _BITS_GW)])
            return carry0

        lax.fori_loop(0, bpw // _BITS_GW, grp, 0)

    return k(keyrep, dep)


def kernel(A):
    dinv = (np.float32(1.0) / jnp.sum(A, axis=1)).reshape(N, 1)
    logP = _compute_logp(A, dinv)

    keys = jax.random.split(jax.random.key(42), NSTEP)
    kd = jax.random.key_data(keys).astype(jnp.uint32)

    def key3(t, woff):
        return jnp.concatenate(
            [kd[t], jnp.array([woff], jnp.uint32)]).reshape(1, 3)

    keyrep = jnp.tile(kd.reshape(NSTEP, 2, 1), (1, 1, 16))
    init_state = jnp.arange(N, dtype=jnp.int32)

    H = W // 2
    states = []
    contribs = []
    bits0 = _sc_bits(keyrep[0], init_state)
    bits1 = _sc_bits(keyrep[1], init_state)
    nxt, ctr = _sample_step(logP, bits0, key3(0, 0))
    states.append(nxt)
    contribs.append(ctr)
    st_a = lax.slice(nxt.reshape(W), (0,), (H,))
    st_b = lax.slice(nxt.reshape(W), (H,), (W,))
    bits = bits1
    sts_a, sts_b, ctrs_a, ctrs_b = [], [], [], []
    for t in range(1, NSTEP):
        rows_a = _sc_gather(logP, st_a)
        rows_b = _sc_gather(logP, st_b)
        nbits = _sc_bits(keyrep[t + 1], st_a) if t + 1 < NSTEP else None
        nxt_a, ctr_a = _sample_step(rows_a, bits, key3(t, 0), 0)
        nxt_b, ctr_b = _sample_step(rows_b, bits, key3(t, H), H // _BW)
        bits = nbits
        sts_a.append(nxt_a)
        sts_b.append(nxt_b)
        ctrs_a.append(ctr_a)
        ctrs_b.append(ctr_b)
        st_a = nxt_a.reshape(H)
        st_b = nxt_b.reshape(H)

    top = jnp.concatenate([states[0][:H]] + sts_a, axis=1)
    bot = jnp.concatenate([states[0][H:]] + sts_b, axis=1)
    init_state = jnp.arange(N, dtype=jnp.int32).reshape(W, 1)
    walks = jnp.concatenate(
        [init_state, jnp.concatenate([top, bot], axis=0)], axis=1)
    ctop = jnp.concatenate([contribs[0][:H]] + ctrs_a, axis=1)
    cbot = jnp.concatenate([contribs[0][H:]] + ctrs_b, axis=1)
    logp = jnp.sum(jnp.concatenate([ctop, cbot], axis=0), axis=1)
    return walks, logp


# speedup vs baseline: 1.1622x; 1.1622x over previous
import functools

import numpy as np
import jax
import jax.numpy as jnp
from jax import lax
from jax.experimental import pallas as pl
from jax.experimental.pallas import tpu as pltpu
from jax.experimental.pallas import tpu_sc as plsc

N = 4096
T = 40
W = 4096
NSTEP = T - 1

_TINY = np.float32(np.finfo(np.float32).tiny)
_RANGE = np.float32(np.float32(1.0) - _TINY)


def _threefry_bits(k0, k1, idx):
    u32 = np.uint32
    ks2 = k0 ^ k1 ^ u32(0x1BD11BDA)

    def rounds(x0, x1, rots):
        for r in rots:
            x0 = x0 + x1
            x1 = (x1 << u32(r)) | (x1 >> u32(32 - r))
            x1 = x0 ^ x1
        return x0, x1

    r1 = (13, 15, 26, 6)
    r2 = (17, 29, 16, 24)
    x0 = jnp.zeros_like(idx) + k0
    x1 = idx + k1
    x0, x1 = rounds(x0, x1, r1)
    x0 = x0 + k1
    x1 = x1 + (ks2 + u32(1))
    x0, x1 = rounds(x0, x1, r2)
    x0 = x0 + ks2
    x1 = x1 + (k0 + u32(2))
    x0, x1 = rounds(x0, x1, r1)
    x0 = x0 + k0
    x1 = x1 + (k1 + u32(3))
    x0, x1 = rounds(x0, x1, r2)
    x0 = x0 + k1
    x1 = x1 + (ks2 + u32(4))
    x0, x1 = rounds(x0, x1, r1)
    x0 = x0 + ks2
    x1 = x1 + (k0 + u32(5))
    return x0 ^ x1


def _gumbel_from_bits(bits):
    fb = (bits >> np.uint32(9)) | np.uint32(0x3F800000)
    f = lax.bitcast_convert_type(fb, jnp.float32) - np.float32(1.0)
    u = jnp.maximum(_TINY, f * _RANGE + _TINY)
    return -jnp.log(-jnp.log(u))


_LOGP_BR = 256


def _logp_body(a_ref, dinv_ref, out_ref):
    out_ref[...] = jnp.log(a_ref[...] * dinv_ref[...] + np.float32(1e-30))


def _compute_logp(A, dinv):
    return pl.pallas_call(
        _logp_body,
        grid=(N // _LOGP_BR,),
        in_specs=[
            pl.BlockSpec((_LOGP_BR, N), lambda b: (b, 0)),
            pl.BlockSpec((_LOGP_BR, 1), lambda b: (b, 0)),
        ],
        out_specs=pl.BlockSpec((_LOGP_BR, N), lambda b: (b, 0)),
        out_shape=jax.ShapeDtypeStruct((N, N), jnp.float32),
    )(A, dinv)


_BW = 128
_CW = 512
_CB = 1024


def _sample_body(key_ref, rows_ref, bits_ref, next_ref, contrib_ref):
    b = pl.program_id(0)
    k0 = key_ref[0, 0]
    k1 = key_ref[0, 1]
    w0 = key_ref[0, 2] + (b * _BW).astype(jnp.uint32)

    run_max = jnp.full((_BW, 1), -jnp.inf, jnp.float32)
    run_arg = jnp.zeros((_BW, 1), jnp.int32)
    run_val = jnp.zeros((_BW, 1), jnp.float32)

    for c in range(N // _CW):
        row = rows_ref[:, c * _CW:(c + 1) * _CW]
        if (c + 1) * _CW <= _CB:
            bits = bits_ref[:, c * _CW:(c + 1) * _CW]
        else:
            ridx = lax.broadcasted_iota(jnp.uint32, (_BW, _CW), 0)
            cidx = lax.broadcasted_iota(jnp.uint32, (_BW, _CW), 1)
            idx = (w0 + ridx) * np.uint32(N) + (np.uint32(c * _CW) + cidx)
            bits = _threefry_bits(k0, k1, idx)
        g = _gumbel_from_bits(bits)
        vals = g + row
        cmax = jnp.max(vals, axis=1, keepdims=True)
        colix = lax.broadcasted_iota(jnp.int32, (_BW, _CW), 1) + np.int32(c * _CW)
        carg = jnp.min(
            jnp.where(vals == cmax, colix, np.int32(2147483647)),
            axis=1, keepdims=True)
        cval = jnp.sum(
            jnp.where(colix == carg, row, np.float32(0.0)),
            axis=1, keepdims=True)
        take = cmax > run_max
        run_arg = jnp.where(take, carg, run_arg)
        run_val = jnp.where(take, cval, run_val)
        run_max = jnp.maximum(run_max, cmax)

    next_ref[...] = run_arg
    contrib_ref[...] = run_val


def _sample_step(rows, bits, key3, wblk=0):
    B = rows.shape[0]
    return pl.pallas_call(
        _sample_body,
        grid=(B // _BW,),
        in_specs=[
            pl.BlockSpec(memory_space=pltpu.SMEM),
            pl.BlockSpec((_BW, N), lambda b: (b, 0)),
            pl.BlockSpec((_BW, _CB), lambda b, _w=wblk: (b + _w, 0)),
        ],
        out_specs=[
            pl.BlockSpec((_BW, 1), lambda b: (b, 0)),
            pl.BlockSpec((_BW, 1), lambda b: (b, 0)),
        ],
        out_shape=[
            jax.ShapeDtypeStruct((B, 1), jnp.int32),
            jax.ShapeDtypeStruct((B, 1), jnp.float32),
        ],
    )(key3, rows, bits)


_SC_NW = 32
_SC_CH = 8


def _sc_gather(table, idx):
    B = idx.shape[0]
    bpw = B // _SC_NW
    nch = bpw // _SC_CH
    mesh = plsc.VectorSubcoreMesh(core_axis_name="c", subcore_axis_name="s")

    @functools.partial(
        pl.kernel,
        mesh=mesh,
        out_type=jax.ShapeDtypeStruct((B, N), jnp.float32),
        scratch_types=[
            pltpu.VMEM((bpw,), jnp.int32),
            pltpu.VMEM((_SC_CH, N), jnp.float32),
            pltpu.VMEM((_SC_CH, N), jnp.float32),
            pltpu.SemaphoreType.DMA,
            pltpu.SemaphoreType.DMA,
        ],
    )
    def k(table_hbm, idx_hbm, out_hbm, idx_v, rows0, rows1, sem0, sem1):
        wid = lax.axis_index("s") * 2 + lax.axis_index("c")
        base = wid * bpw
        pltpu.sync_copy(idx_hbm.at[pl.ds(base, bpw)], idx_v)
        bufs = (rows0, rows1)
        sems = (sem0, sem1)

        def dma(i):
            return pltpu.make_async_copy(
                table_hbm.at[idx_v.at[pl.ds(i * _SC_CH, _SC_CH)]],
                bufs[i % 2], sems[i % 2])

        dma(0).start()
        for i in range(nch):
            if i + 1 < nch:
                dma(i + 1).start()
            dma(i).wait()
            pltpu.sync_copy(
                bufs[i % 2],
                out_hbm.at[pl.ds(base + i * _SC_CH, _SC_CH)])

    return k(table, idx)


_BITS_GW = 16


def _sc_bits(keyrep, dep):
    mesh = plsc.VectorSubcoreMesh(core_axis_name="c", subcore_axis_name="s")
    bpw = W // _SC_NW

    @functools.partial(
        pl.kernel,
        mesh=mesh,
        out_type=jax.ShapeDtypeStruct((W, _CB), jnp.uint32),
        scratch_types=[
            pltpu.VMEM((2, 16), jnp.uint32),
            pltpu.VMEM((_BITS_GW, _CB), jnp.uint32),
            pltpu.VMEM((16,), jnp.int32),
        ],
    )
    def k(keyrep_hbm, dep_hbm, out_hbm, kv, buf, depv):
        wid = lax.axis_index("s") * 2 + lax.axis_index("c")
        pltpu.sync_copy(dep_hbm.at[pl.ds(0, 16)], depv)
        pltpu.sync_copy(keyrep_hbm, kv)
        k0 = kv[0, :]
        k1 = kv[1, :]
        iota = lax.iota(jnp.int32, 16).astype(jnp.uint32)
        base_w = wid * bpw

        def grp(gi, carry0):
            def wloop(wl, carry1):
                w = base_w + gi * _BITS_GW + wl
                rowbase = (w * N).astype(jnp.uint32)

                def cloop(cv, carry2):
                    j0 = cv * 16
                    idx = jnp.full((16,), rowbase + j0.astype(jnp.uint32),
                                   jnp.uint32) + iota
                    buf[wl, pl.ds(j0, 16)] = _threefry_bits(k0, k1, idx)
                    return carry2

                lax.fori_loop(0, _CB // 16, cloop, 0)
                return carry1

            lax.fori_loop(0, _BITS_GW, wloop, 0)
            pltpu.sync_copy(
                buf, out_hbm.at[pl.ds(base_w + gi * _BITS_GW, _BITS_GW)])
            return carry0

        lax.fori_loop(0, bpw // _BITS_GW, grp, 0)

    return k(keyrep, dep)


def kernel(A):
    dinv = (np.float32(1.0) / jnp.sum(A, axis=1)).reshape(N, 1)
    logP = _compute_logp(A, dinv)

    keys = jax.random.split(jax.random.key(42), NSTEP)
    kd = jax.random.key_data(keys).astype(jnp.uint32)

    def key3(t, woff):
        return jnp.concatenate(
            [kd[t], jnp.array([woff], jnp.uint32)]).reshape(1, 3)

    keyrep = jnp.tile(kd.reshape(NSTEP, 2, 1), (1, 1, 16))
    init_state = jnp.arange(N, dtype=jnp.int32)

    H = W // 2
    states = []
    contribs = []
    bits0 = _sc_bits(keyrep[0], init_state)
    bits1 = _sc_bits(keyrep[1], init_state)
    nxt, ctr = _sample_step(logP, bits0, key3(0, 0))
    states.append(nxt)
    contribs.append(ctr)
    st_a = lax.slice(nxt.reshape(W), (0,), (H,))
    st_b = lax.slice(nxt.reshape(W), (H,), (W,))
    bits = bits1
    sts_a, sts_b, ctrs_a, ctrs_b = [], [], [], []
    for t in range(1, NSTEP):
        rows_a = _sc_gather(logP, st_a)
        rows_b = _sc_gather(logP, st_b)
        nbits = _sc_bits(keyrep[t + 1], st_a) if t + 1 < NSTEP else None
        nxt_a, ctr_a = _sample_step(rows_a, bits, key3(t, 0), 0)
        nxt_b, ctr_b = _sample_step(rows_b, bits, key3(t, H), H // _BW)
        bits = nbits
        sts_a.append(nxt_a)
        sts_b.append(nxt_b)
        ctrs_a.append(ctr_a)
        ctrs_b.append(ctr_b)
        st_a = nxt_a.reshape(H)
        st_b = nxt_b.reshape(H)

    top = jnp.concatenate([states[0][:H]] + sts_a, axis=1)
    bot = jnp.concatenate([states[0][H:]] + sts_b, axis=1)
    init_state = jnp.arange(N, dtype=jnp.int32).reshape(W, 1)
    walks = jnp.concatenate(
        [init_state, jnp.concatenate([top, bot], axis=0)], axis=1)
    ctop = jnp.concatenate([contribs[0][:H]] + ctrs_a, axis=1)
    cbot = jnp.concatenate([contribs[0][H:]] + ctrs_b, axis=1)
    logp = jnp.sum(jnp.concatenate([ctop, cbot], axis=0), axis=1)
    return walks, logp
